# baseline (speedup 1.0000x reference)
import functools

import jax
import jax.numpy as jnp
from jax import lax
from jax.experimental import pallas as pl
from jax.experimental.pallas import tpu as pltpu
from jax.experimental.pallas import tpu_sc as plsc

NC = 2
NS = 16
NW = NC * NS
K2 = 256
DW = 16
DG = 64
BN = 512


def _cdiv(a, b):
  return (a + b - 1) // b


def _sc_degree(np_, ch):
  mesh = plsc.VectorSubcoreMesh(core_axis_name="c", subcore_axis_name="s")
  rpt = np_ // NS

  @functools.partial(
      pl.kernel,
      out_type=jax.ShapeDtypeStruct((NC, np_, DW), jnp.float32),
      mesh=mesh,
      compiler_params=pltpu.CompilerParams(use_tc_tiling_on_sc=False),
      scratch_types=[
          pltpu.VMEM_SHARED((np_, DW), jnp.float32),
          pltpu.VMEM((K2, DW), jnp.float32),
          pltpu.VMEM((2, 2, K2), jnp.int32),
      ] + [pltpu.SemaphoreType.DMA] * 4,
  )
  def deg_kernel(sd_hbm, ones_hbm, zeros_hbm, out_hbm, acc, ones_v, idxv,
                 *sems):
    isems = sems[0:2]
    ssems = sems[2:4]
    c = lax.axis_index("c")
    s = lax.axis_index("s")
    wid = c * NS + s
    pltpu.sync_copy(zeros_hbm.at[pl.ds(s * rpt, rpt)],
                    acc.at[pl.ds(s * rpt, rpt)])
    pltpu.sync_copy(ones_hbm, ones_v)
    plsc.subcore_barrier()

    @pl.loop(0, ch // 2)
    def _(t):
      idd = [
          pltpu.async_copy(sd_hbm.at[wid, 2 * t + b], idxv.at[b], isems[b])
          for b in range(2)
      ]
      sd = []
      for b in range(2):
        idd[b].wait()
        sd.append(
            pltpu.async_copy(ones_v, acc.at[idxv.at[b, 1]], ssems[b],
                             add=True))
      for b in range(2):
        sd[b].wait()

    plsc.subcore_barrier()
    pltpu.sync_copy(acc.at[pl.ds(s * rpt, rpt)],
                    out_hbm.at[c, pl.ds(s * rpt, rpt)])

  return deg_kernel


def _sc_aggregate(np_, d, cha, chb, chmax, dtype):
  del chmax
  mesh = plsc.VectorSubcoreMesh(core_axis_name="c", subcore_axis_name="s")
  rpt = np_ // NS

  @functools.partial(
      pl.kernel,
      out_type=jax.ShapeDtypeStruct((NC, np_, d), dtype),
      mesh=mesh,
      compiler_params=pltpu.CompilerParams(use_tc_tiling_on_sc=False),
      scratch_types=[
          pltpu.VMEM_SHARED((np_, d), dtype),
          pltpu.VMEM((2, K2, d), dtype),
          pltpu.VMEM((2, 2, K2), jnp.int32),
      ] + [pltpu.SemaphoreType.DMA] * 6,
  )
  def agg_kernel(table_hbm, sd_hbm, zeros_hbm, out_hbm, acc, rows, idxv,
                 *sems):
    isems = sems[0:2]
    gsems = sems[2:4]
    ssems = sems[4:6]
    c = lax.axis_index("c")
    s = lax.axis_index("s")
    wid = c * NS + s
    nch = jnp.where(c == 0, cha, chb)
    pltpu.sync_copy(zeros_hbm.at[pl.ds(s * rpt, rpt)],
                    acc.at[pl.ds(s * rpt, rpt)])
    plsc.subcore_barrier()

    @pl.loop(0, nch // 2)
    def _(t):
      idd = [
          pltpu.async_copy(sd_hbm.at[wid, 2 * t + b], idxv.at[b], isems[b])
          for b in range(2)
      ]
      gd = []
      for b in range(2):
        idd[b].wait()
        gd.append(
            pltpu.async_copy(table_hbm.at[idxv.at[b, 0]], rows.at[b],
                             gsems[b]))
      sd = []
      for b in range(2):
        gd[b].wait()
        sd.append(
            pltpu.async_copy(rows.at[b], acc.at[idxv.at[b, 1]], ssems[b],
                             add=True))
      for b in range(2):
        sd[b].wait()

    plsc.subcore_barrier()
    pltpu.sync_copy(acc.at[pl.ds(s * rpt, rpt)],
                    out_hbm.at[c, pl.ds(s * rpt, rpt)])

  return agg_kernel


def _tc_scale_body(x_ref, t0_ref, t1_ref, xs_ref, dinv_ref):
  deg = 1.0 + t0_ref[0, :, 0:1] + t1_ref[0, :, 0:1]
  dinv = lax.rsqrt(deg)
  xs_ref[...] = (x_ref[...] * dinv).astype(xs_ref.dtype)
  dinv_ref[...] = dinv


def _tc_dense_body(p0_ref, p1_ref, xs_ref, dinv_ref, w1_ref, b1_ref, w2_ref,
                   ys_ref):
  d = dinv_ref[...]
  agg = (p0_ref[0].astype(jnp.float32) + p1_ref[0].astype(jnp.float32) +
         xs_ref[...].astype(jnp.float32))
  z = d * agg
  h = jnp.dot(z, w1_ref[...], preferred_element_type=jnp.float32)
  h = jnp.maximum(h + b1_ref[...], 0.0)
  y = jnp.dot(h, w2_ref[...], preferred_element_type=jnp.float32)
  ys_ref[...] = (d * y).astype(ys_ref.dtype)


def _tc_softmax_body(c_valid, q0_ref, q1_ref, ys_ref, dinv_ref, b2_ref, o_ref):
  agg = (q0_ref[0].astype(jnp.float32) + q1_ref[0].astype(jnp.float32) +
         ys_ref[...].astype(jnp.float32))
  u = dinv_ref[...] * agg + b2_ref[...]
  col = lax.broadcasted_iota(jnp.int32, u.shape, 1)
  valid = col < c_valid
  um = jnp.where(valid, u, -jnp.inf)
  mx = jnp.max(um, axis=1, keepdims=True)
  ex = jnp.where(valid, jnp.exp(u - mx), 0.0)
  o_ref[...] = (u - mx) - jnp.log(jnp.sum(ex, axis=1, keepdims=True))


def _row_spec(d):
  return pl.BlockSpec((BN, d), lambda i: (i, 0))


def _slab_spec(slab, d):
  return pl.BlockSpec((1, BN, d), lambda i, _s=slab: (_s, i, 0))


def _full_spec(r, c):
  return pl.BlockSpec((r, c), lambda i: (0, 0))


def kernel(x, edge_index, W1, b1, W2, b2):
  n, f_in = x.shape
  hid = W1.shape[1]
  c_out = W2.shape[1]
  e = edge_index.shape[1]

  np_ = _cdiv(n, NS * BN) * NS * BN
  ch = _cdiv(_cdiv(e, NW * K2), 2) * 2
  ep = NW * ch * K2
  d2 = _cdiv(c_out, 64) * 64

  src = edge_index[0].astype(jnp.int32)
  dst = edge_index[1].astype(jnp.int32)
  pad = ep - e
  src_f = jnp.concatenate([src, jnp.zeros((pad,), jnp.int32)])
  dst_f = jnp.concatenate([dst, jnp.full((pad,), n, jnp.int32)])
  sd_p = jnp.stack(
      [src_f.reshape(NW, ch, K2), dst_f.reshape(NW, ch, K2)], axis=2)

  cha = max(2, (_cdiv(2 * ch, 4) // 2) * 2 - 2)
  chb = 2 * ch - cha
  chmax = max(cha, chb)
  ea = NS * cha * K2
  sd0 = jnp.stack([src_f[:ea].reshape(NS, cha, K2),
                   dst_f[:ea].reshape(NS, cha, K2)], axis=2)
  sd0 = jnp.pad(sd0, ((0, 0), (0, chmax - cha), (0, 0), (0, 0)))
  sd1 = jnp.stack([src_f[ea:].reshape(NS, chb, K2),
                   dst_f[ea:].reshape(NS, chb, K2)], axis=2)
  sd1 = jnp.pad(sd1, ((0, 0), (0, chmax - chb), (0, 0), (0, 0)))
  sd_q = jnp.concatenate([sd0, sd1], axis=0)

  x_p = jnp.zeros((np_, f_in), jnp.float32).at[:n].set(x)
  ones_rows = jnp.zeros((K2, DW), jnp.float32).at[:, 0].set(1.0)
  zeros_deg = jnp.zeros((np_, DW), jnp.float32)
  zeros_g = jnp.zeros((np_, f_in), jnp.bfloat16)
  zeros_c = jnp.zeros((np_, d2), jnp.bfloat16)
  w2_p = jnp.zeros((hid, d2), jnp.float32).at[:, :c_out].set(W2)
  b1_r = b1.reshape(1, hid)
  b2_p = jnp.zeros((1, d2), jnp.float32).at[0, :c_out].set(b2)

  deg_t = _sc_degree(np_, ch)(sd_p, ones_rows, zeros_deg)

  grid = (np_ // BN,)
  xs, dinv = pl.pallas_call(
      _tc_scale_body,
      grid=grid,
      in_specs=[_row_spec(f_in), _slab_spec(0, DW), _slab_spec(1, DW)],
      out_specs=[_row_spec(f_in), _row_spec(1)],
      out_shape=[
          jax.ShapeDtypeStruct((np_, f_in), jnp.bfloat16),
          jax.ShapeDtypeStruct((np_, 1), jnp.float32),
      ],
  )(x_p, deg_t, deg_t)

  p_t = _sc_aggregate(np_, f_in, cha, chb, chmax, jnp.bfloat16)(
      xs, sd_q, zeros_g)

  ys = pl.pallas_call(
      _tc_dense_body,
      grid=grid,
      in_specs=[
          _slab_spec(0, f_in), _slab_spec(1, f_in), _row_spec(f_in),
          _row_spec(1),
          _full_spec(f_in, hid), _full_spec(1, hid), _full_spec(hid, d2),
      ],
      out_specs=_row_spec(d2),
      out_shape=jax.ShapeDtypeStruct((np_, d2), jnp.bfloat16),
  )(p_t, p_t, xs, dinv, W1, b1_r, w2_p)

  q_t = _sc_aggregate(np_, d2, cha, chb, chmax, jnp.bfloat16)(
      ys, sd_q, zeros_c)

  o = pl.pallas_call(
      functools.partial(_tc_softmax_body, c_out),
      grid=grid,
      in_specs=[
          _slab_spec(0, d2), _slab_spec(1, d2), _row_spec(d2), _row_spec(1),
          _full_spec(1, d2),
      ],
      out_specs=_row_spec(d2),
      out_shape=jax.ShapeDtypeStruct((np_, d2), jnp.float32),
  )(q_t, q_t, ys, dinv, b2_p)

  return o[:n, :c_out]


# speedup vs baseline: 1.0444x; 1.0444x over previous
import functools

import jax
import jax.numpy as jnp
from jax import lax
from jax.experimental import pallas as pl
from jax.experimental.pallas import tpu as pltpu
from jax.experimental.pallas import tpu_sc as plsc

NC = 2
NS = 16
NW = NC * NS
K2 = 256
DW = 16
DG = 64
BN = 512


def _cdiv(a, b):
  return (a + b - 1) // b


def _sc_degree(np_, ch):
  mesh = plsc.VectorSubcoreMesh(core_axis_name="c", subcore_axis_name="s")
  rpt = np_ // NS

  @functools.partial(
      pl.kernel,
      out_type=jax.ShapeDtypeStruct((NC, np_, DW), jnp.float32),
      mesh=mesh,
      compiler_params=pltpu.CompilerParams(use_tc_tiling_on_sc=False,
                                           skip_device_barrier=True),
      scratch_types=[
          pltpu.VMEM_SHARED((np_, DW), jnp.float32),
          pltpu.VMEM((K2, DW), jnp.float32),
          pltpu.VMEM((2, 2, K2), jnp.int32),
      ] + [pltpu.SemaphoreType.DMA] * 4,
  )
  def deg_kernel(sd_hbm, ones_hbm, zeros_hbm, out_hbm, acc, ones_v, idxv,
                 *sems):
    isems = sems[0:2]
    ssems = sems[2:4]
    c = lax.axis_index("c")
    s = lax.axis_index("s")
    wid = c * NS + s
    pltpu.sync_copy(zeros_hbm.at[pl.ds(s * rpt, rpt)],
                    acc.at[pl.ds(s * rpt, rpt)])
    pltpu.sync_copy(ones_hbm, ones_v)
    plsc.subcore_barrier()

    @pl.loop(0, ch // 2)
    def _(t):
      idd = [
          pltpu.async_copy(sd_hbm.at[wid, 2 * t + b], idxv.at[b], isems[b])
          for b in range(2)
      ]
      sd = []
      for b in range(2):
        idd[b].wait()
        sd.append(
            pltpu.async_copy(ones_v, acc.at[idxv.at[b, 1]], ssems[b],
                             add=True))
      for b in range(2):
        sd[b].wait()

    plsc.subcore_barrier()
    pltpu.sync_copy(acc.at[pl.ds(s * rpt, rpt)],
                    out_hbm.at[c, pl.ds(s * rpt, rpt)])

  return deg_kernel


def _sc_aggregate(np_, d, cha, chb, chmax, dtype):
  del chmax
  mesh = plsc.VectorSubcoreMesh(core_axis_name="c", subcore_axis_name="s")
  rpt = np_ // NS

  @functools.partial(
      pl.kernel,
      out_type=jax.ShapeDtypeStruct((NC, np_, d), dtype),
      mesh=mesh,
      compiler_params=pltpu.CompilerParams(use_tc_tiling_on_sc=False,
                                           skip_device_barrier=True),
      scratch_types=[
          pltpu.VMEM_SHARED((np_, d), dtype),
          pltpu.VMEM((2, K2, d), dtype),
          pltpu.VMEM((2, 2, K2), jnp.int32),
      ] + [pltpu.SemaphoreType.DMA] * 6,
  )
  def agg_kernel(table_hbm, sd_hbm, zeros_hbm, out_hbm, acc, rows, idxv,
                 *sems):
    isems = sems[0:2]
    gsems = sems[2:4]
    ssems = sems[4:6]
    c = lax.axis_index("c")
    s = lax.axis_index("s")
    wid = c * NS + s
    nch = jnp.where(c == 0, cha, chb)
    pltpu.sync_copy(zeros_hbm.at[pl.ds(s * rpt, rpt)],
                    acc.at[pl.ds(s * rpt, rpt)])
    plsc.subcore_barrier()

    @pl.loop(0, nch // 2)
    def _(t):
      idd = [
          pltpu.async_copy(sd_hbm.at[wid, 2 * t + b], idxv.at[b], isems[b])
          for b in range(2)
      ]
      gd = []
      for b in range(2):
        idd[b].wait()
        gd.append(
            pltpu.async_copy(table_hbm.at[idxv.at[b, 0]], rows.at[b],
                             gsems[b]))
      sd = []
      for b in range(2):
        gd[b].wait()
        sd.append(
            pltpu.async_copy(rows.at[b], acc.at[idxv.at[b, 1]], ssems[b],
                             add=True))
      for b in range(2):
        sd[b].wait()

    plsc.subcore_barrier()
    pltpu.sync_copy(acc.at[pl.ds(s * rpt, rpt)],
                    out_hbm.at[c, pl.ds(s * rpt, rpt)])

  return agg_kernel


def _tc_scale_body(x_ref, t0_ref, t1_ref, xs_ref, dinv_ref):
  deg = 1.0 + t0_ref[:, 0:1] + t1_ref[:, 0:1]
  dinv = lax.rsqrt(deg)
  xs_ref[...] = (x_ref[...] * dinv).astype(xs_ref.dtype)
  dinv_ref[...] = dinv


def _tc_dense_body(p0_ref, p1_ref, xs_ref, dinv_ref, w1_ref, b1_ref, w2_ref,
                   ys_ref):
  d = dinv_ref[...]
  agg = (p0_ref[...].astype(jnp.float32) + p1_ref[...].astype(jnp.float32) +
         xs_ref[...].astype(jnp.float32))
  z = d * agg
  h = jnp.dot(z, w1_ref[...], preferred_element_type=jnp.float32)
  h = jnp.maximum(h + b1_ref[...], 0.0)
  y = jnp.dot(h, w2_ref[...], preferred_element_type=jnp.float32)
  ys_ref[...] = (d * y).astype(ys_ref.dtype)


def _tc_softmax_body(c_valid, q0_ref, q1_ref, ys_ref, dinv_ref, b2_ref, o_ref):
  agg = (q0_ref[...].astype(jnp.float32) + q1_ref[...].astype(jnp.float32) +
         ys_ref[...].astype(jnp.float32))
  u = dinv_ref[...] * agg + b2_ref[...]
  col = lax.broadcasted_iota(jnp.int32, u.shape, 1)
  valid = col < c_valid
  um = jnp.where(valid, u, -jnp.inf)
  mx = jnp.max(um, axis=1, keepdims=True)
  ex = jnp.where(valid, jnp.exp(u - mx), 0.0)
  o_ref[...] = (u - mx) - jnp.log(jnp.sum(ex, axis=1, keepdims=True))


def _row_spec(d):
  return pl.BlockSpec((BN, d), lambda i: (i, 0))


def _full_spec(r, c):
  return pl.BlockSpec((r, c), lambda i: (0, 0))


def kernel(x, edge_index, W1, b1, W2, b2):
  n, f_in = x.shape
  hid = W1.shape[1]
  c_out = W2.shape[1]
  e = edge_index.shape[1]

  np_ = _cdiv(n, NS * BN) * NS * BN
  ch = _cdiv(_cdiv(e, NW * K2), 2) * 2
  ep = NW * ch * K2
  d2 = _cdiv(c_out, 64) * 64

  src = edge_index[0].astype(jnp.int32)
  dst = edge_index[1].astype(jnp.int32)
  pad = ep - e
  src_f = jnp.concatenate([src, jnp.zeros((pad,), jnp.int32)])
  dst_f = jnp.concatenate([dst, jnp.full((pad,), n, jnp.int32)])
  sd_p = jnp.stack(
      [src_f.reshape(NW, ch, K2), dst_f.reshape(NW, ch, K2)], axis=2)

  cha = max(2, (_cdiv(2 * ch, 4) // 2) * 2 - 2)
  chb = 2 * ch - cha
  chmax = max(cha, chb)
  ea = NS * cha * K2
  sd0 = jnp.stack([src_f[:ea].reshape(NS, cha, K2),
                   dst_f[:ea].reshape(NS, cha, K2)], axis=2)
  sd0 = jnp.pad(sd0, ((0, 0), (0, chmax - cha), (0, 0), (0, 0)))
  sd1 = jnp.stack([src_f[ea:].reshape(NS, chb, K2),
                   dst_f[ea:].reshape(NS, chb, K2)], axis=2)
  sd1 = jnp.pad(sd1, ((0, 0), (0, chmax - chb), (0, 0), (0, 0)))
  sd_q = jnp.concatenate([sd0, sd1], axis=0)

  x_p = jnp.zeros((np_, f_in), jnp.float32).at[:n].set(x)
  ones_rows = jnp.zeros((K2, DW), jnp.float32).at[:, 0].set(1.0)
  zeros_deg = jnp.zeros((np_, DW), jnp.float32)
  zeros_g = jnp.zeros((np_, f_in), jnp.bfloat16)
  zeros_c = jnp.zeros((np_, d2), jnp.bfloat16)
  w2_p = jnp.zeros((hid, d2), jnp.float32).at[:, :c_out].set(W2)
  b1_r = b1.reshape(1, hid)
  b2_p = jnp.zeros((1, d2), jnp.float32).at[0, :c_out].set(b2)

  deg_t = _sc_degree(np_, ch)(sd_p, ones_rows, zeros_deg)

  grid = (np_ // BN,)
  xs, dinv = pl.pallas_call(
      _tc_scale_body,
      grid=grid,
      in_specs=[_row_spec(f_in), _row_spec(DW), _row_spec(DW)],
      out_specs=[_row_spec(f_in), _row_spec(1)],
      out_shape=[
          jax.ShapeDtypeStruct((np_, f_in), jnp.bfloat16),
          jax.ShapeDtypeStruct((np_, 1), jnp.float32),
      ],
  )(x_p, deg_t[0], deg_t[1])

  p_t = _sc_aggregate(np_, f_in, cha, chb, chmax, jnp.bfloat16)(
      xs, sd_q, zeros_g)

  ys = pl.pallas_call(
      _tc_dense_body,
      grid=grid,
      in_specs=[
          _row_spec(f_in), _row_spec(f_in), _row_spec(f_in), _row_spec(1),
          _full_spec(f_in, hid), _full_spec(1, hid), _full_spec(hid, d2),
      ],
      out_specs=_row_spec(d2),
      out_shape=jax.ShapeDtypeStruct((np_, d2), jnp.bfloat16),
  )(p_t[0], p_t[1], xs, dinv, W1, b1_r, w2_p)

  q_t = _sc_aggregate(np_, d2, cha, chb, chmax, jnp.bfloat16)(
      ys, sd_q, zeros_c)

  o = pl.pallas_call(
      functools.partial(_tc_softmax_body, c_out),
      grid=grid,
      in_specs=[
          _row_spec(d2), _row_spec(d2), _row_spec(d2), _row_spec(1),
          _full_spec(1, d2),
      ],
      out_specs=_row_spec(d2),
      out_shape=jax.ShapeDtypeStruct((np_, d2), jnp.float32),
  )(q_t[0], q_t[1], ys, dinv, b2_p)

  return o[:n, :c_out]


# speedup vs baseline: 1.0447x; 1.0003x over previous
import functools

import jax
import jax.numpy as jnp
from jax import lax
from jax.experimental import pallas as pl
from jax.experimental.pallas import tpu as pltpu
from jax.experimental.pallas import tpu_sc as plsc

NC = 2
NS = 16
NW = NC * NS
K2 = 256
DW = 16
DG = 64
BN = 512


def _cdiv(a, b):
  return (a + b - 1) // b


def _sc_degree(np_, ch):
  mesh = plsc.VectorSubcoreMesh(core_axis_name="c", subcore_axis_name="s")
  rpt = np_ // NS

  @functools.partial(
      pl.kernel,
      out_type=jax.ShapeDtypeStruct((NC, np_, DW), jnp.float32),
      mesh=mesh,
      compiler_params=pltpu.CompilerParams(use_tc_tiling_on_sc=False),
      scratch_types=[
          pltpu.VMEM_SHARED((np_, DW), jnp.float32),
          pltpu.VMEM((K2, DW), jnp.float32),
          pltpu.VMEM((2, 2, K2), jnp.int32),
      ] + [pltpu.SemaphoreType.DMA] * 4,
  )
  def deg_kernel(sd_hbm, ones_hbm, zeros_hbm, out_hbm, acc, ones_v, idxv,
                 *sems):
    isems = sems[0:2]
    ssems = sems[2:4]
    c = lax.axis_index("c")
    s = lax.axis_index("s")
    wid = c * NS + s
    pltpu.sync_copy(zeros_hbm.at[pl.ds(s * rpt, rpt)],
                    acc.at[pl.ds(s * rpt, rpt)])
    pltpu.sync_copy(ones_hbm, ones_v)
    plsc.subcore_barrier()

    @pl.loop(0, ch // 2)
    def _(t):
      idd = [
          pltpu.async_copy(sd_hbm.at[wid, 2 * t + b], idxv.at[b], isems[b])
          for b in range(2)
      ]
      sd = []
      for b in range(2):
        idd[b].wait()
        sd.append(
            pltpu.async_copy(ones_v, acc.at[idxv.at[b, 1]], ssems[b],
                             add=True))
      for b in range(2):
        sd[b].wait()

    plsc.subcore_barrier()
    pltpu.sync_copy(acc.at[pl.ds(s * rpt, rpt)],
                    out_hbm.at[c, pl.ds(s * rpt, rpt)])

  return deg_kernel


def _sc_aggregate(np_, d, cha, chb, chmax, dtype):
  del chmax
  mesh = plsc.VectorSubcoreMesh(core_axis_name="c", subcore_axis_name="s")
  rpt = np_ // NS

  @functools.partial(
      pl.kernel,
      out_type=jax.ShapeDtypeStruct((NC, np_, d), dtype),
      mesh=mesh,
      compiler_params=pltpu.CompilerParams(use_tc_tiling_on_sc=False),
      scratch_types=[
          pltpu.VMEM_SHARED((np_, d), dtype),
          pltpu.VMEM((2, K2, d), dtype),
          pltpu.VMEM((2, 2, K2), jnp.int32),
      ] + [pltpu.SemaphoreType.DMA] * 6,
  )
  def agg_kernel(table_hbm, sd_hbm, zeros_hbm, out_hbm, acc, rows, idxv,
                 *sems):
    isems = sems[0:2]
    gsems = sems[2:4]
    ssems = sems[4:6]
    c = lax.axis_index("c")
    s = lax.axis_index("s")
    wid = c * NS + s
    nch = jnp.where(c == 0, cha, chb)
    pltpu.sync_copy(zeros_hbm.at[pl.ds(s * rpt, rpt)],
                    acc.at[pl.ds(s * rpt, rpt)])
    plsc.subcore_barrier()

    @pl.loop(0, nch // 2)
    def _(t):
      idd = [
          pltpu.async_copy(sd_hbm.at[wid, 2 * t + b], idxv.at[b], isems[b])
          for b in range(2)
      ]
      gd = []
      for b in range(2):
        idd[b].wait()
        gd.append(
            pltpu.async_copy(table_hbm.at[idxv.at[b, 0]], rows.at[b],
                             gsems[b]))
      sd = []
      for b in range(2):
        gd[b].wait()
        sd.append(
            pltpu.async_copy(rows.at[b], acc.at[idxv.at[b, 1]], ssems[b],
                             add=True))
      for b in range(2):
        sd[b].wait()

    plsc.subcore_barrier()
    pltpu.sync_copy(acc.at[pl.ds(s * rpt, rpt)],
                    out_hbm.at[c, pl.ds(s * rpt, rpt)])

  return agg_kernel


def _tc_scale_body(x_ref, t0_ref, t1_ref, xs_ref, dinv_ref):
  deg = 1.0 + t0_ref[:, 0:1] + t1_ref[:, 0:1]
  dinv = lax.rsqrt(deg)
  xs_ref[...] = (x_ref[...] * dinv).astype(xs_ref.dtype)
  dinv_ref[...] = dinv


def _tc_dense_body(p0_ref, p1_ref, xs_ref, dinv_ref, w1_ref, b1_ref, w2_ref,
                   ys_ref):
  d = dinv_ref[...]
  agg = (p0_ref[...].astype(jnp.float32) + p1_ref[...].astype(jnp.float32) +
         xs_ref[...].astype(jnp.float32))
  z = d * agg
  h = jnp.dot(z, w1_ref[...], preferred_element_type=jnp.float32)
  h = jnp.maximum(h + b1_ref[...], 0.0)
  y = jnp.dot(h, w2_ref[...], preferred_element_type=jnp.float32)
  ys_ref[...] = (d * y).astype(ys_ref.dtype)


def _tc_softmax_body(c_valid, q0_ref, q1_ref, ys_ref, dinv_ref, b2_ref, o_ref):
  agg = (q0_ref[...].astype(jnp.float32) + q1_ref[...].astype(jnp.float32) +
         ys_ref[...].astype(jnp.float32))
  u = dinv_ref[...] * agg + b2_ref[...]
  col = lax.broadcasted_iota(jnp.int32, u.shape, 1)
  valid = col < c_valid
  um = jnp.where(valid, u, -jnp.inf)
  mx = jnp.max(um, axis=1, keepdims=True)
  ex = jnp.where(valid, jnp.exp(u - mx), 0.0)
  o_ref[...] = (u - mx) - jnp.log(jnp.sum(ex, axis=1, keepdims=True))


def _row_spec(d):
  return pl.BlockSpec((BN, d), lambda i: (i, 0))


def _full_spec(r, c):
  return pl.BlockSpec((r, c), lambda i: (0, 0))


def kernel(x, edge_index, W1, b1, W2, b2):
  n, f_in = x.shape
  hid = W1.shape[1]
  c_out = W2.shape[1]
  e = edge_index.shape[1]

  np_ = _cdiv(n, NS * BN) * NS * BN
  ch = _cdiv(_cdiv(e, NW * K2), 2) * 2
  ep = NW * ch * K2
  d2 = _cdiv(c_out, 64) * 64

  src = edge_index[0].astype(jnp.int32)
  dst = edge_index[1].astype(jnp.int32)
  pad = ep - e
  src_f = jnp.concatenate([src, jnp.zeros((pad,), jnp.int32)])
  dst_f = jnp.concatenate([dst, jnp.full((pad,), n, jnp.int32)])
  sd_p = jnp.stack(
      [src_f.reshape(NW, ch, K2), dst_f.reshape(NW, ch, K2)], axis=2)

  cha = max(2, (_cdiv(2 * ch, 4) // 2) * 2 - 2)
  chb = 2 * ch - cha
  chmax = max(cha, chb)
  ea = NS * cha * K2
  sd0 = jnp.stack([src_f[:ea].reshape(NS, cha, K2),
                   dst_f[:ea].reshape(NS, cha, K2)], axis=2)
  sd0 = jnp.pad(sd0, ((0, 0), (0, chmax - cha), (0, 0), (0, 0)))
  sd1 = jnp.stack([src_f[ea:].reshape(NS, chb, K2),
                   dst_f[ea:].reshape(NS, chb, K2)], axis=2)
  sd1 = jnp.pad(sd1, ((0, 0), (0, chmax - chb), (0, 0), (0, 0)))
  sd_q = jnp.concatenate([sd0, sd1], axis=0)

  x_p = jnp.zeros((np_, f_in), jnp.float32).at[:n].set(x)
  ones_rows = jnp.zeros((K2, DW), jnp.float32).at[:, 0].set(1.0)
  zeros_deg = jnp.zeros((np_, DW), jnp.float32)
  zeros_g = jnp.zeros((np_, f_in), jnp.bfloat16)
  zeros_c = jnp.zeros((np_, d2), jnp.bfloat16)
  w2_p = jnp.zeros((hid, d2), jnp.float32).at[:, :c_out].set(W2)
  b1_r = b1.reshape(1, hid)
  b2_p = jnp.zeros((1, d2), jnp.float32).at[0, :c_out].set(b2)

  deg_t = _sc_degree(np_, ch)(sd_p, ones_rows, zeros_deg)

  grid = (np_ // BN,)
  xs, dinv = pl.pallas_call(
      _tc_scale_body,
      grid=grid,
      in_specs=[_row_spec(f_in), _row_spec(DW), _row_spec(DW)],
      out_specs=[_row_spec(f_in), _row_spec(1)],
      out_shape=[
          jax.ShapeDtypeStruct((np_, f_in), jnp.bfloat16),
          jax.ShapeDtypeStruct((np_, 1), jnp.float32),
      ],
  )(x_p, deg_t[0], deg_t[1])

  p_t = _sc_aggregate(np_, f_in, cha, chb, chmax, jnp.bfloat16)(
      xs, sd_q, zeros_g)

  ys = pl.pallas_call(
      _tc_dense_body,
      grid=grid,
      in_specs=[
          _row_spec(f_in), _row_spec(f_in), _row_spec(f_in), _row_spec(1),
          _full_spec(f_in, hid), _full_spec(1, hid), _full_spec(hid, d2),
      ],
      out_specs=_row_spec(d2),
      out_shape=jax.ShapeDtypeStruct((np_, d2), jnp.bfloat16),
  )(p_t[0], p_t[1], xs, dinv, W1, b1_r, w2_p)

  q_t = _sc_aggregate(np_, d2, cha, chb, chmax, jnp.bfloat16)(
      ys, sd_q, zeros_c)

  o = pl.pallas_call(
      functools.partial(_tc_softmax_body, c_out),
      grid=grid,
      in_specs=[
          _row_spec(d2), _row_spec(d2), _row_spec(d2), _row_spec(1),
          _full_spec(1, d2),
      ],
      out_specs=_row_spec(d2),
      out_shape=jax.ShapeDtypeStruct((np_, d2), jnp.float32),
  )(q_t[0], q_t[1], ys, dinv, b2_p)

  return o[:n, :c_out]
